# baseline (speedup 1.0000x reference)
import numpy as np
import jax
import jax.numpy as jnp
from jax.experimental import pallas as pl
from jax.experimental.pallas import tpu as pltpu

S, B = 96, 16
IN_SIZE = 1380
D_E = 64
GH = 64
N_SPK = 2
WP, WF = 10, 10
N_CLASSES = 6
NF = 2 * D_E
D_H = NF + GH
SB = S * B
PAD = WP

_idx = np.arange(S)
_inv_past = (1.0 / np.maximum(np.minimum(_idx, WP), 1.0)).astype(np.float32)
_inv_fut = (1.0 / np.maximum(np.minimum(S - 1 - _idx, WF), 1.0)).astype(np.float32)
_INV0 = np.repeat(_inv_past[:, None], B, axis=1)
_INV2 = np.repeat(_inv_fut[:, None], B, axis=1)

_f32 = jnp.float32


def _mm(a, b):
    return jnp.dot(a, b, preferred_element_type=_f32)


def _cell(g, c, hid):
    i_ = jax.nn.sigmoid(g[:, 0 * hid:1 * hid])
    f_ = jax.nn.sigmoid(g[:, 1 * hid:2 * hid])
    g_ = jnp.tanh(g[:, 2 * hid:3 * hid])
    o_ = jax.nn.sigmoid(g[:, 3 * hid:4 * hid])
    c2 = f_ * c + i_ * g_
    return o_ * jnp.tanh(c2), c2


def _lstm_multi(streams, gp_f, gp_b):
    offs = []
    off = 0
    for (x2d, hid, wf, uf, bf, wb, ub, bb, out_ref) in streams:
        H4 = 4 * hid
        gp_f[:, :, off:off + H4] = (_mm(x2d, wf) + bf).reshape(S, B, H4)
        gp_b[:, :, off:off + H4] = (_mm(x2d, wb) + bb).reshape(S, B, H4)
        offs.append(off)
        off += H4

    def step(t, carry):
        new = []
        for i, (x2d, hid, wf, uf, bf, wb, ub, bb, out_ref) in enumerate(streams):
            H4 = 4 * hid
            o = offs[i]
            hf, cf, hb, cb = carry[4 * i:4 * i + 4]
            ggf = gp_f[t, :, o:o + H4] + _mm(hf, uf)
            ggb = gp_b[S - 1 - t, :, o:o + H4] + _mm(hb, ub)
            hf2, cf2 = _cell(ggf, cf, hid)
            hb2, cb2 = _cell(ggb, cb, hid)
            out_ref[t, :, :hid] = hf2
            out_ref[S - 1 - t, :, hid:2 * hid] = hb2
            new += [hf2, cf2, hb2, cb2]
        return tuple(new)

    z = [jnp.zeros((B, st[1]), _f32) for st in streams for _ in range(4)]
    jax.lax.fori_loop(0, S, step, tuple(z), unroll=96)


def _softmax_rows(x):
    m = jnp.max(x, axis=-1, keepdims=True)
    e = jnp.exp(x - m)
    return e / jnp.sum(e, axis=-1, keepdims=True)


def _base_body(ins, feats_ref, gp_f, gp_b, tmp128):
    def ld(k):
        return ins[k][...]

    def lw(px):
        return (ld(px + 'wf'), ld(px + 'uf'), ld(px + 'bf'),
                ld(px + 'wb'), ld(px + 'ub'), ld(px + 'bb'))

    x2d = ld('x').reshape(SB, IN_SIZE)
    _lstm_multi([(x2d, D_E) + lw('b1_') + (tmp128,)], gp_f, gp_b)
    _lstm_multi([(tmp128[...].reshape(SB, NF), D_E) + lw('b2_') + (feats_ref,)],
                gp_f, gp_b)


def _graph_body(ins, hid_ref, gp_f, gp_b, cat1, cat2, sq1, sq2, m1, m2,
                pband, pmask):
    def ld(k):
        return ins[k][...]

    def lw(px):
        return (ld(px + 'wf'), ld(px + 'uf'), ld(px + 'bf'),
                ld(px + 'wb'), ld(px + 'ub'), ld(px + 'bb'))

    def fill_pband(val3, w):
        pband[0:PAD, :, :w] = jnp.zeros((PAD, B, w), _f32)
        pband[PAD + S:PAD + S + WF, :, :w] = jnp.zeros((WF, B, w), _f32)
        pband[PAD:PAD + S, :, :w] = val3

    def band_sums(w):
        sumL = pband[PAD - 1:PAD - 1 + S, :, :w]
        for d0 in range(2, WP + 1):
            sumL = sumL + pband[PAD - d0:PAD - d0 + S, :, :w]
        sumU = pband[PAD + 1:PAD + 1 + S, :, :w]
        for d0 in range(2, WF + 1):
            sumU = sumU + pband[PAD + d0:PAD + d0 + S, :, :w]
        return sumL, sumU

    f3 = ld('feats')
    f2 = f3.reshape(SB, NF)

    def aggregate(px, speaker, cat_ref):
        g = lambda k: ld(px + k)
        out2 = _mm(f2, g('root')) + g('rgcn_b')
        if not speaker:
            fill_pband(f3, NF)
            sumL, sumU = band_sums(NF)
            agg0 = sumL * ld('inv0')[:, :, None]
            agg2 = sumU * ld('inv2')[:, :, None]
            out2 = out2 + _mm(agg0.reshape(SB, NF), g('W0'))
            out2 = out2 + _mm(f2, g('W1'))
            out2 = out2 + _mm(agg2.reshape(SB, NF), g('W2'))
        else:
            qm = (ld('qm0'), ld('qm1'))
            for a in range(N_SPK):
                qa = qm[a]
                ya3 = f3 * qa[:, :, None]
                fill_pband(ya3, NF)
                sumL, sumU = band_sums(NF)
                ssum = sumL + sumU + ya3
                pmask[0:PAD, :] = jnp.zeros((PAD, B), _f32)
                pmask[PAD + S:PAD + S + WF, :] = jnp.zeros((WF, B), _f32)
                pmask[PAD:PAD + S, :] = qa
                cnt = qa
                for d0 in range(1, WP + 1):
                    cnt = cnt + pmask[PAD - d0:PAD - d0 + S, :]
                for d0 in range(1, WF + 1):
                    cnt = cnt + pmask[PAD + d0:PAD + d0 + S, :]
                smean = ssum / jnp.maximum(cnt, 1.0)[:, :, None]
                for b_ in range(N_SPK):
                    sel = (smean * qm[b_][:, :, None]).reshape(SB, NF)
                    out2 = out2 + _mm(sel, g('W%d%d' % (a, b_)))
        o3 = out2.reshape(S, B, GH)
        fill_pband(o3, GH)
        sumL, sumU = band_sums(GH)
        aggF = (sumL + sumU + o3).reshape(SB, GH)
        gc2 = _mm(aggF, g('gc_wrel')) + g('gc_brel') + _mm(out2, g('gc_wroot'))
        cat_ref[:, :, :NF] = f3
        cat_ref[:, :, NF:] = gc2.reshape(S, B, GH)

    aggregate('gt_', False, cat1)
    aggregate('gs_', True, cat2)

    _lstm_multi([
        (cat1[...].reshape(SB, D_H), D_H) + lw('gt_l1_') + (sq1,),
        (cat2[...].reshape(SB, D_H), D_H) + lw('gs_l1_') + (sq2,),
    ], gp_f, gp_b)
    _lstm_multi([
        (sq1[...].reshape(SB, 2 * D_H), D_H) + lw('gt_l2_') + (m1,),
        (sq2[...].reshape(SB, 2 * D_H), D_H) + lw('gs_l2_') + (m2,),
    ], gp_f, gp_b)

    sq1[...] = (_mm(m1[...].reshape(SB, 2 * D_H), ld('gt_att_wt'))
                + ld('gt_att_b')).reshape(S, B, 2 * D_H)
    sq2[...] = (_mm(m2[...].reshape(SB, 2 * D_H), ld('gs_att_wt'))
                + ld('gs_att_b')).reshape(S, B, 2 * D_H)
    lin1, lb1 = ld('gt_lin_wt'), ld('gt_lin_b')
    lin2, lb2 = ld('gs_lin_wt'), ld('gs_lin_b')
    for b in range(B):
        hbs = []
        for (mref, gref, lwt, lbv) in ((m1, sq1, lin1, lb1), (m2, sq2, lin2, lb2)):
            Mb = mref[:, b, :]
            Gb = gref[:, b, :]
            sc = jnp.tanh(jax.lax.dot_general(
                Gb, Mb, (((1,), (1,)), ((), ())), preferred_element_type=_f32))
            al = _softmax_rows(sc)
            att = _mm(al, Mb)
            hbs.append(jnp.maximum(_mm(att, lwt) + lbv, 0.0))
        hid_ref[:, b, :] = hbs[0] + hbs[1]


def _final_body(ins, logp_ref, rec_ref):
    def ld(k):
        return ins[k][...]
    h2d = ld('hidden').reshape(SB, D_H)
    logp_ref[...] = (_mm(h2d, ld('smax_wt')) + ld('smax_b')).reshape(S, B, N_CLASSES)
    rec_ref[...] = (_mm(h2d, ld('rec_wt')) + ld('rec_b')).reshape(S, B, IN_SIZE)


def _lstm_ins(px, lp):
    return {
        px + 'wf': lp['W_ih_f'].T, px + 'uf': lp['W_hh_f'].T,
        px + 'bf': lp['b_f'].reshape(1, -1),
        px + 'wb': lp['W_ih_b'].T, px + 'ub': lp['W_hh_b'].T,
        px + 'bb': lp['b_b'].reshape(1, -1),
    }


def _graph_ins(px, gp, speaker):
    d = {
        px + 'root': gp['rgcn_root'], px + 'rgcn_b': gp['rgcn_b'].reshape(1, -1),
        px + 'gc_wrel': gp['gc_W_rel'], px + 'gc_brel': gp['gc_b_rel'].reshape(1, -1),
        px + 'gc_wroot': gp['gc_W_root'],
        px + 'att_wt': gp['att_W'].T, px + 'att_b': gp['att_b'].reshape(1, -1),
        px + 'lin_wt': gp['lin_W'].T, px + 'lin_b': gp['lin_b'].reshape(1, -1),
    }
    W = gp['rgcn_W']
    if speaker:
        for a in range(N_SPK):
            for b_ in range(N_SPK):
                d[px + 'W%d%d' % (a, b_)] = W[a * N_SPK + b_]
    else:
        for r in range(3):
            d[px + 'W%d' % r] = W[r]
    d.update(_lstm_ins(px + 'l1_', gp['lstm'][0]))
    d.update(_lstm_ins(px + 'l2_', gp['lstm'][1]))
    return d


_CP = pltpu.CompilerParams(vmem_limit_bytes=100 * 1024 * 1024)


def kernel(inputfeats, qmask, umask, seq_lengths, params):
    base_ins = {'x': inputfeats[0]}
    base_ins.update(_lstm_ins('b1_', params['lstm_base'][0]))
    base_ins.update(_lstm_ins('b2_', params['lstm_base'][1]))
    feats = pl.pallas_call(
        _base_body,
        out_shape=jax.ShapeDtypeStruct((S, B, NF), _f32),
        scratch_shapes=[
            pltpu.VMEM((S, B, 4 * D_E), _f32),
            pltpu.VMEM((S, B, 4 * D_E), _f32),
            pltpu.VMEM((S, B, NF), _f32),
        ],
        compiler_params=_CP)(base_ins)

    graph_ins = {
        'feats': feats,
        'qm0': qmask[:, :, 0], 'qm1': qmask[:, :, 1],
        'inv0': jnp.asarray(_INV0), 'inv2': jnp.asarray(_INV2),
    }
    graph_ins.update(_graph_ins('gt_', params['g_temporal'], False))
    graph_ins.update(_graph_ins('gs_', params['g_speaker'], True))
    hidden = pl.pallas_call(
        _graph_body,
        out_shape=jax.ShapeDtypeStruct((S, B, D_H), _f32),
        scratch_shapes=[
            pltpu.VMEM((S, B, 2 * 4 * D_H), _f32),
            pltpu.VMEM((S, B, 2 * 4 * D_H), _f32),
            pltpu.VMEM((S, B, D_H), _f32),
            pltpu.VMEM((S, B, D_H), _f32),
            pltpu.VMEM((S, B, 2 * D_H), _f32),
            pltpu.VMEM((S, B, 2 * D_H), _f32),
            pltpu.VMEM((S, B, 2 * D_H), _f32),
            pltpu.VMEM((S, B, 2 * D_H), _f32),
            pltpu.VMEM((S + WP + WF, B, NF), _f32),
            pltpu.VMEM((S + WP + WF, B), _f32),
        ],
        compiler_params=_CP)(graph_ins)

    final_ins = {
        'hidden': hidden,
        'smax_wt': params['smax_W'].T, 'smax_b': params['smax_b'].reshape(1, -1),
        'rec_wt': params['rec_W'].T, 'rec_b': params['rec_b'].reshape(1, -1),
    }
    logp, rec = pl.pallas_call(
        _final_body,
        out_shape=[
            jax.ShapeDtypeStruct((S, B, N_CLASSES), _f32),
            jax.ShapeDtypeStruct((S, B, IN_SIZE), _f32),
        ],
        compiler_params=_CP)(final_ins)
    return logp, rec, hidden


# speedup vs baseline: 1.0021x; 1.0021x over previous
import numpy as np
import jax
import jax.numpy as jnp
from jax.experimental import pallas as pl
from jax.experimental.pallas import tpu as pltpu

S, B = 96, 16
IN_SIZE = 1380
D_E = 64
GH = 64
N_SPK = 2
WP, WF = 10, 10
N_CLASSES = 6
NF = 2 * D_E
D_H = NF + GH
SB = S * B
PAD = WP

_idx = np.arange(S)
_inv_past = (1.0 / np.maximum(np.minimum(_idx, WP), 1.0)).astype(np.float32)
_inv_fut = (1.0 / np.maximum(np.minimum(S - 1 - _idx, WF), 1.0)).astype(np.float32)
_INV0 = np.repeat(_inv_past[:, None], B, axis=1)
_INV2 = np.repeat(_inv_fut[:, None], B, axis=1)

_f32 = jnp.float32


def _mm(a, b):
    return jnp.dot(a, b, preferred_element_type=_f32)


def _cell(g, c, hid):
    i_ = jax.nn.sigmoid(g[:, 0 * hid:1 * hid])
    f_ = jax.nn.sigmoid(g[:, 1 * hid:2 * hid])
    g_ = jnp.tanh(g[:, 2 * hid:3 * hid])
    o_ = jax.nn.sigmoid(g[:, 3 * hid:4 * hid])
    c2 = f_ * c + i_ * g_
    return o_ * jnp.tanh(c2), c2


def _lstm_multi(streams, gp_f, gp_b):
    offs = []
    off = 0
    for (x2d, hid, wf, uf, bf, wb, ub, bb, out_ref) in streams:
        H4 = 4 * hid
        gp_f[:, :, off:off + H4] = (_mm(x2d, wf) + bf).reshape(S, B, H4)
        gp_b[:, :, off:off + H4] = (_mm(x2d, wb) + bb).reshape(S, B, H4)
        offs.append(off)
        off += H4

    def step(t, carry):
        new = []
        for i, (x2d, hid, wf, uf, bf, wb, ub, bb, out_ref) in enumerate(streams):
            H4 = 4 * hid
            o = offs[i]
            hf, cf, hb, cb = carry[4 * i:4 * i + 4]
            ggf = gp_f[t, :, o:o + H4] + _mm(hf, uf)
            ggb = gp_b[S - 1 - t, :, o:o + H4] + _mm(hb, ub)
            hf2, cf2 = _cell(ggf, cf, hid)
            hb2, cb2 = _cell(ggb, cb, hid)
            out_ref[t, :, :hid] = hf2
            out_ref[S - 1 - t, :, hid:2 * hid] = hb2
            new += [hf2, cf2, hb2, cb2]
        return tuple(new)

    z = [jnp.zeros((B, st[1]), _f32) for st in streams for _ in range(4)]
    jax.lax.fori_loop(0, S, step, tuple(z), unroll=32)


def _softmax_rows(x):
    m = jnp.max(x, axis=-1, keepdims=True)
    e = jnp.exp(x - m)
    return e / jnp.sum(e, axis=-1, keepdims=True)


def _base_body(ins, feats_ref, gp_f, gp_b, tmp128):
    def ld(k):
        return ins[k][...]

    def lw(px):
        return (ld(px + 'wf'), ld(px + 'uf'), ld(px + 'bf'),
                ld(px + 'wb'), ld(px + 'ub'), ld(px + 'bb'))

    x2d = ld('x').reshape(SB, IN_SIZE)
    _lstm_multi([(x2d, D_E) + lw('b1_') + (tmp128,)], gp_f, gp_b)
    _lstm_multi([(tmp128[...].reshape(SB, NF), D_E) + lw('b2_') + (feats_ref,)],
                gp_f, gp_b)


def _graph_body(ins, hid_ref, gp_f, gp_b, cat1, cat2, sq1, sq2, m1, m2,
                pband, pmask):
    def ld(k):
        return ins[k][...]

    def lw(px):
        return (ld(px + 'wf'), ld(px + 'uf'), ld(px + 'bf'),
                ld(px + 'wb'), ld(px + 'ub'), ld(px + 'bb'))

    def fill_pband(val3, w):
        pband[0:PAD, :, :w] = jnp.zeros((PAD, B, w), _f32)
        pband[PAD + S:PAD + S + WF, :, :w] = jnp.zeros((WF, B, w), _f32)
        pband[PAD:PAD + S, :, :w] = val3

    def band_sums(w):
        sumL = pband[PAD - 1:PAD - 1 + S, :, :w]
        for d0 in range(2, WP + 1):
            sumL = sumL + pband[PAD - d0:PAD - d0 + S, :, :w]
        sumU = pband[PAD + 1:PAD + 1 + S, :, :w]
        for d0 in range(2, WF + 1):
            sumU = sumU + pband[PAD + d0:PAD + d0 + S, :, :w]
        return sumL, sumU

    f3 = ld('feats')
    f2 = f3.reshape(SB, NF)

    def aggregate(px, speaker, cat_ref):
        g = lambda k: ld(px + k)
        out2 = _mm(f2, g('root')) + g('rgcn_b')
        if not speaker:
            fill_pband(f3, NF)
            sumL, sumU = band_sums(NF)
            agg0 = sumL * ld('inv0')[:, :, None]
            agg2 = sumU * ld('inv2')[:, :, None]
            out2 = out2 + _mm(agg0.reshape(SB, NF), g('W0'))
            out2 = out2 + _mm(f2, g('W1'))
            out2 = out2 + _mm(agg2.reshape(SB, NF), g('W2'))
        else:
            qm = (ld('qm0'), ld('qm1'))
            for a in range(N_SPK):
                qa = qm[a]
                ya3 = f3 * qa[:, :, None]
                fill_pband(ya3, NF)
                sumL, sumU = band_sums(NF)
                ssum = sumL + sumU + ya3
                pmask[0:PAD, :] = jnp.zeros((PAD, B), _f32)
                pmask[PAD + S:PAD + S + WF, :] = jnp.zeros((WF, B), _f32)
                pmask[PAD:PAD + S, :] = qa
                cnt = qa
                for d0 in range(1, WP + 1):
                    cnt = cnt + pmask[PAD - d0:PAD - d0 + S, :]
                for d0 in range(1, WF + 1):
                    cnt = cnt + pmask[PAD + d0:PAD + d0 + S, :]
                smean = ssum / jnp.maximum(cnt, 1.0)[:, :, None]
                for b_ in range(N_SPK):
                    sel = (smean * qm[b_][:, :, None]).reshape(SB, NF)
                    out2 = out2 + _mm(sel, g('W%d%d' % (a, b_)))
        o3 = out2.reshape(S, B, GH)
        fill_pband(o3, GH)
        sumL, sumU = band_sums(GH)
        aggF = (sumL + sumU + o3).reshape(SB, GH)
        gc2 = _mm(aggF, g('gc_wrel')) + g('gc_brel') + _mm(out2, g('gc_wroot'))
        cat_ref[:, :, :NF] = f3
        cat_ref[:, :, NF:] = gc2.reshape(S, B, GH)

    aggregate('gt_', False, cat1)
    aggregate('gs_', True, cat2)

    _lstm_multi([
        (cat1[...].reshape(SB, D_H), D_H) + lw('gt_l1_') + (sq1,),
        (cat2[...].reshape(SB, D_H), D_H) + lw('gs_l1_') + (sq2,),
    ], gp_f, gp_b)
    _lstm_multi([
        (sq1[...].reshape(SB, 2 * D_H), D_H) + lw('gt_l2_') + (m1,),
        (sq2[...].reshape(SB, 2 * D_H), D_H) + lw('gs_l2_') + (m2,),
    ], gp_f, gp_b)

    sq1[...] = (_mm(m1[...].reshape(SB, 2 * D_H), ld('gt_att_wt'))
                + ld('gt_att_b')).reshape(S, B, 2 * D_H)
    sq2[...] = (_mm(m2[...].reshape(SB, 2 * D_H), ld('gs_att_wt'))
                + ld('gs_att_b')).reshape(S, B, 2 * D_H)
    lin1, lb1 = ld('gt_lin_wt'), ld('gt_lin_b')
    lin2, lb2 = ld('gs_lin_wt'), ld('gs_lin_b')
    for b in range(B):
        hbs = []
        for (mref, gref, lwt, lbv) in ((m1, sq1, lin1, lb1), (m2, sq2, lin2, lb2)):
            Mb = mref[:, b, :]
            Gb = gref[:, b, :]
            sc = jnp.tanh(jax.lax.dot_general(
                Gb, Mb, (((1,), (1,)), ((), ())), preferred_element_type=_f32))
            al = _softmax_rows(sc)
            att = _mm(al, Mb)
            hbs.append(jnp.maximum(_mm(att, lwt) + lbv, 0.0))
        hid_ref[:, b, :] = hbs[0] + hbs[1]


def _final_body(ins, logp_ref, rec_ref):
    def ld(k):
        return ins[k][...]
    h2d = ld('hidden').reshape(SB, D_H)
    logp_ref[...] = (_mm(h2d, ld('smax_wt')) + ld('smax_b')).reshape(S, B, N_CLASSES)
    rec_ref[...] = (_mm(h2d, ld('rec_wt')) + ld('rec_b')).reshape(S, B, IN_SIZE)


def _lstm_ins(px, lp):
    return {
        px + 'wf': lp['W_ih_f'].T, px + 'uf': lp['W_hh_f'].T,
        px + 'bf': lp['b_f'].reshape(1, -1),
        px + 'wb': lp['W_ih_b'].T, px + 'ub': lp['W_hh_b'].T,
        px + 'bb': lp['b_b'].reshape(1, -1),
    }


def _graph_ins(px, gp, speaker):
    d = {
        px + 'root': gp['rgcn_root'], px + 'rgcn_b': gp['rgcn_b'].reshape(1, -1),
        px + 'gc_wrel': gp['gc_W_rel'], px + 'gc_brel': gp['gc_b_rel'].reshape(1, -1),
        px + 'gc_wroot': gp['gc_W_root'],
        px + 'att_wt': gp['att_W'].T, px + 'att_b': gp['att_b'].reshape(1, -1),
        px + 'lin_wt': gp['lin_W'].T, px + 'lin_b': gp['lin_b'].reshape(1, -1),
    }
    W = gp['rgcn_W']
    if speaker:
        for a in range(N_SPK):
            for b_ in range(N_SPK):
                d[px + 'W%d%d' % (a, b_)] = W[a * N_SPK + b_]
    else:
        for r in range(3):
            d[px + 'W%d' % r] = W[r]
    d.update(_lstm_ins(px + 'l1_', gp['lstm'][0]))
    d.update(_lstm_ins(px + 'l2_', gp['lstm'][1]))
    return d


_CP = pltpu.CompilerParams(vmem_limit_bytes=100 * 1024 * 1024)


def kernel(inputfeats, qmask, umask, seq_lengths, params):
    base_ins = {'x': inputfeats[0]}
    base_ins.update(_lstm_ins('b1_', params['lstm_base'][0]))
    base_ins.update(_lstm_ins('b2_', params['lstm_base'][1]))
    feats = pl.pallas_call(
        _base_body,
        out_shape=jax.ShapeDtypeStruct((S, B, NF), _f32),
        scratch_shapes=[
            pltpu.VMEM((S, B, 4 * D_E), _f32),
            pltpu.VMEM((S, B, 4 * D_E), _f32),
            pltpu.VMEM((S, B, NF), _f32),
        ],
        compiler_params=_CP)(base_ins)

    graph_ins = {
        'feats': feats,
        'qm0': qmask[:, :, 0], 'qm1': qmask[:, :, 1],
        'inv0': jnp.asarray(_INV0), 'inv2': jnp.asarray(_INV2),
    }
    graph_ins.update(_graph_ins('gt_', params['g_temporal'], False))
    graph_ins.update(_graph_ins('gs_', params['g_speaker'], True))
    hidden = pl.pallas_call(
        _graph_body,
        out_shape=jax.ShapeDtypeStruct((S, B, D_H), _f32),
        scratch_shapes=[
            pltpu.VMEM((S, B, 2 * 4 * D_H), _f32),
            pltpu.VMEM((S, B, 2 * 4 * D_H), _f32),
            pltpu.VMEM((S, B, D_H), _f32),
            pltpu.VMEM((S, B, D_H), _f32),
            pltpu.VMEM((S, B, 2 * D_H), _f32),
            pltpu.VMEM((S, B, 2 * D_H), _f32),
            pltpu.VMEM((S, B, 2 * D_H), _f32),
            pltpu.VMEM((S, B, 2 * D_H), _f32),
            pltpu.VMEM((S + WP + WF, B, NF), _f32),
            pltpu.VMEM((S + WP + WF, B), _f32),
        ],
        compiler_params=_CP)(graph_ins)

    final_ins = {
        'hidden': hidden,
        'smax_wt': params['smax_W'].T, 'smax_b': params['smax_b'].reshape(1, -1),
        'rec_wt': params['rec_W'].T, 'rec_b': params['rec_b'].reshape(1, -1),
    }
    logp, rec = pl.pallas_call(
        _final_body,
        out_shape=[
            jax.ShapeDtypeStruct((S, B, N_CLASSES), _f32),
            jax.ShapeDtypeStruct((S, B, IN_SIZE), _f32),
        ],
        compiler_params=_CP)(final_ins)
    return logp, rec, hidden


# speedup vs baseline: 1.0280x; 1.0259x over previous
import numpy as np
import jax
import jax.numpy as jnp
from jax.experimental import pallas as pl
from jax.experimental.pallas import tpu as pltpu

S, B = 96, 16
IN_SIZE = 1380
D_E = 64
GH = 64
N_SPK = 2
WP, WF = 10, 10
N_CLASSES = 6
NF = 2 * D_E
D_H = NF + GH
SB = S * B
PAD = WP

_idx = np.arange(S)
_inv_past = (1.0 / np.maximum(np.minimum(_idx, WP), 1.0)).astype(np.float32)
_inv_fut = (1.0 / np.maximum(np.minimum(S - 1 - _idx, WF), 1.0)).astype(np.float32)
_INV0 = np.repeat(_inv_past[:, None], B, axis=1)
_INV2 = np.repeat(_inv_fut[:, None], B, axis=1)

_f32 = jnp.float32


def _mm(a, b):
    return jnp.dot(a, b, preferred_element_type=_f32)


def _cell(g, c, hid):
    i_ = jax.nn.sigmoid(g[:, 0 * hid:1 * hid])
    f_ = jax.nn.sigmoid(g[:, 1 * hid:2 * hid])
    g_ = jnp.tanh(g[:, 2 * hid:3 * hid])
    o_ = jax.nn.sigmoid(g[:, 3 * hid:4 * hid])
    c2 = f_ * c + i_ * g_
    return o_ * jnp.tanh(c2), c2


def _lstm_multi(streams, gp_f, gp_b):
    offs = []
    off = 0
    for (x2d, hid, wfb, bfb, uf, ub, out_ref) in streams:
        H4 = 4 * hid
        g2 = _mm(x2d, wfb) + bfb
        gp_f[:, :, off:off + H4] = g2[:, :H4].reshape(S, B, H4)
        gp_b[:, :, off:off + H4] = g2[:, H4:].reshape(S, B, H4)
        offs.append(off)
        off += H4

    def step(t, carry):
        new = []
        for i, (x2d, hid, wfb, bfb, uf, ub, out_ref) in enumerate(streams):
            H4 = 4 * hid
            o = offs[i]
            hf, cf, hb, cb = carry[4 * i:4 * i + 4]
            ggf = gp_f[t, :, o:o + H4] + _mm(hf, uf)
            ggb = gp_b[S - 1 - t, :, o:o + H4] + _mm(hb, ub)
            hf2, cf2 = _cell(ggf, cf, hid)
            hb2, cb2 = _cell(ggb, cb, hid)
            out_ref[t, :, :hid] = hf2
            out_ref[S - 1 - t, :, hid:2 * hid] = hb2
            new += [hf2, cf2, hb2, cb2]
        return tuple(new)

    z = [jnp.zeros((B, st[1]), _f32) for st in streams for _ in range(4)]
    jax.lax.fori_loop(0, S, step, tuple(z), unroll=32)


def _softmax_rows(x):
    m = jnp.max(x, axis=-1, keepdims=True)
    e = jnp.exp(x - m)
    return e / jnp.sum(e, axis=-1, keepdims=True)


def _base_body(ins, feats_ref, gp_f, gp_b, tmp128):
    def ld(k):
        return ins[k][...]

    def lw(px):
        return (ld(px + 'wfb'), ld(px + 'bfb'), ld(px + 'uf'), ld(px + 'ub'))

    x2d = ld('x').reshape(SB, IN_SIZE)
    _lstm_multi([(x2d, D_E) + lw('b1_') + (tmp128,)], gp_f, gp_b)
    _lstm_multi([(tmp128[...].reshape(SB, NF), D_E) + lw('b2_') + (feats_ref,)],
                gp_f, gp_b)


def _graph_body(ins, hid_ref, gp_f, gp_b, cat1, cat2, sq1, sq2, m1, m2,
                pband, pmask):
    def ld(k):
        return ins[k][...]

    def lw(px):
        return (ld(px + 'wfb'), ld(px + 'bfb'), ld(px + 'uf'), ld(px + 'ub'))

    def fill_pband(val3, w):
        pband[0:PAD, :, :w] = jnp.zeros((PAD, B, w), _f32)
        pband[PAD + S:PAD + S + WF, :, :w] = jnp.zeros((WF, B, w), _f32)
        pband[PAD:PAD + S, :, :w] = val3

    def band_sums(w):
        sumL = pband[PAD - 1:PAD - 1 + S, :, :w]
        for d0 in range(2, WP + 1):
            sumL = sumL + pband[PAD - d0:PAD - d0 + S, :, :w]
        sumU = pband[PAD + 1:PAD + 1 + S, :, :w]
        for d0 in range(2, WF + 1):
            sumU = sumU + pband[PAD + d0:PAD + d0 + S, :, :w]
        return sumL, sumU

    f3 = ld('feats')
    f2 = f3.reshape(SB, NF)

    def aggregate(px, speaker, cat_ref):
        g = lambda k: ld(px + k)
        out2 = _mm(f2, g('root')) + g('rgcn_b')
        if not speaker:
            fill_pband(f3, NF)
            sumL, sumU = band_sums(NF)
            agg0 = sumL * ld('inv0')[:, :, None]
            agg2 = sumU * ld('inv2')[:, :, None]
            out2 = out2 + _mm(agg0.reshape(SB, NF), g('W0'))
            out2 = out2 + _mm(f2, g('W1'))
            out2 = out2 + _mm(agg2.reshape(SB, NF), g('W2'))
        else:
            qm = (ld('qm0'), ld('qm1'))
            for a in range(N_SPK):
                qa = qm[a]
                ya3 = f3 * qa[:, :, None]
                fill_pband(ya3, NF)
                sumL, sumU = band_sums(NF)
                ssum = sumL + sumU + ya3
                pmask[0:PAD, :] = jnp.zeros((PAD, B), _f32)
                pmask[PAD + S:PAD + S + WF, :] = jnp.zeros((WF, B), _f32)
                pmask[PAD:PAD + S, :] = qa
                cnt = qa
                for d0 in range(1, WP + 1):
                    cnt = cnt + pmask[PAD - d0:PAD - d0 + S, :]
                for d0 in range(1, WF + 1):
                    cnt = cnt + pmask[PAD + d0:PAD + d0 + S, :]
                smean = ssum / jnp.maximum(cnt, 1.0)[:, :, None]
                for b_ in range(N_SPK):
                    sel = (smean * qm[b_][:, :, None]).reshape(SB, NF)
                    out2 = out2 + _mm(sel, g('W%d%d' % (a, b_)))
        o3 = out2.reshape(S, B, GH)
        fill_pband(o3, GH)
        sumL, sumU = band_sums(GH)
        aggF = (sumL + sumU + o3).reshape(SB, GH)
        gc2 = _mm(aggF, g('gc_wrel')) + g('gc_brel') + _mm(out2, g('gc_wroot'))
        cat_ref[:, :, :NF] = f3
        cat_ref[:, :, NF:] = gc2.reshape(S, B, GH)

    aggregate('gt_', False, cat1)
    aggregate('gs_', True, cat2)

    _lstm_multi([
        (cat1[...].reshape(SB, D_H), D_H) + lw('gt_l1_') + (sq1,),
        (cat2[...].reshape(SB, D_H), D_H) + lw('gs_l1_') + (sq2,),
    ], gp_f, gp_b)
    _lstm_multi([
        (sq1[...].reshape(SB, 2 * D_H), D_H) + lw('gt_l2_') + (m1,),
        (sq2[...].reshape(SB, 2 * D_H), D_H) + lw('gs_l2_') + (m2,),
    ], gp_f, gp_b)

    sq1[...] = (_mm(m1[...].reshape(SB, 2 * D_H), ld('gt_att_wt'))
                + ld('gt_att_b')).reshape(S, B, 2 * D_H)
    sq2[...] = (_mm(m2[...].reshape(SB, 2 * D_H), ld('gs_att_wt'))
                + ld('gs_att_b')).reshape(S, B, 2 * D_H)
    lin1, lb1 = ld('gt_lin_wt'), ld('gt_lin_b')
    lin2, lb2 = ld('gs_lin_wt'), ld('gs_lin_b')
    for b in range(B):
        hbs = []
        for (mref, gref, lwt, lbv) in ((m1, sq1, lin1, lb1), (m2, sq2, lin2, lb2)):
            Mb = mref[:, b, :]
            Gb = gref[:, b, :]
            sc = jnp.tanh(jax.lax.dot_general(
                Gb, Mb, (((1,), (1,)), ((), ())), preferred_element_type=_f32))
            al = _softmax_rows(sc)
            att = _mm(al, Mb)
            hbs.append(jnp.maximum(_mm(att, lwt) + lbv, 0.0))
        hid_ref[:, b, :] = hbs[0] + hbs[1]


def _final_body(ins, logp_ref, rec_ref):
    def ld(k):
        return ins[k][...]
    h2d = ld('hidden').reshape(SB, D_H)
    logp_ref[...] = (_mm(h2d, ld('smax_wt')) + ld('smax_b')).reshape(S, B, N_CLASSES)
    rec_ref[...] = (_mm(h2d, ld('rec_wt')) + ld('rec_b')).reshape(S, B, IN_SIZE)


def _lstm_ins(px, lp):
    return {
        px + 'wfb': jnp.concatenate([lp['W_ih_f'].T, lp['W_ih_b'].T], axis=1),
        px + 'bfb': jnp.concatenate([lp['b_f'], lp['b_b']]).reshape(1, -1),
        px + 'uf': lp['W_hh_f'].T, px + 'ub': lp['W_hh_b'].T,
    }


def _graph_ins(px, gp, speaker):
    d = {
        px + 'root': gp['rgcn_root'], px + 'rgcn_b': gp['rgcn_b'].reshape(1, -1),
        px + 'gc_wrel': gp['gc_W_rel'], px + 'gc_brel': gp['gc_b_rel'].reshape(1, -1),
        px + 'gc_wroot': gp['gc_W_root'],
        px + 'att_wt': gp['att_W'].T, px + 'att_b': gp['att_b'].reshape(1, -1),
        px + 'lin_wt': gp['lin_W'].T, px + 'lin_b': gp['lin_b'].reshape(1, -1),
    }
    W = gp['rgcn_W']
    if speaker:
        for a in range(N_SPK):
            for b_ in range(N_SPK):
                d[px + 'W%d%d' % (a, b_)] = W[a * N_SPK + b_]
    else:
        for r in range(3):
            d[px + 'W%d' % r] = W[r]
    d.update(_lstm_ins(px + 'l1_', gp['lstm'][0]))
    d.update(_lstm_ins(px + 'l2_', gp['lstm'][1]))
    return d


_CP = pltpu.CompilerParams(vmem_limit_bytes=100 * 1024 * 1024)


def kernel(inputfeats, qmask, umask, seq_lengths, params):
    base_ins = {'x': inputfeats[0]}
    base_ins.update(_lstm_ins('b1_', params['lstm_base'][0]))
    base_ins.update(_lstm_ins('b2_', params['lstm_base'][1]))
    feats = pl.pallas_call(
        _base_body,
        out_shape=jax.ShapeDtypeStruct((S, B, NF), _f32),
        scratch_shapes=[
            pltpu.VMEM((S, B, 4 * D_E), _f32),
            pltpu.VMEM((S, B, 4 * D_E), _f32),
            pltpu.VMEM((S, B, NF), _f32),
        ],
        compiler_params=_CP)(base_ins)

    graph_ins = {
        'feats': feats,
        'qm0': qmask[:, :, 0], 'qm1': qmask[:, :, 1],
        'inv0': jnp.asarray(_INV0), 'inv2': jnp.asarray(_INV2),
    }
    graph_ins.update(_graph_ins('gt_', params['g_temporal'], False))
    graph_ins.update(_graph_ins('gs_', params['g_speaker'], True))
    hidden = pl.pallas_call(
        _graph_body,
        out_shape=jax.ShapeDtypeStruct((S, B, D_H), _f32),
        scratch_shapes=[
            pltpu.VMEM((S, B, 2 * 4 * D_H), _f32),
            pltpu.VMEM((S, B, 2 * 4 * D_H), _f32),
            pltpu.VMEM((S, B, D_H), _f32),
            pltpu.VMEM((S, B, D_H), _f32),
            pltpu.VMEM((S, B, 2 * D_H), _f32),
            pltpu.VMEM((S, B, 2 * D_H), _f32),
            pltpu.VMEM((S, B, 2 * D_H), _f32),
            pltpu.VMEM((S, B, 2 * D_H), _f32),
            pltpu.VMEM((S + WP + WF, B, NF), _f32),
            pltpu.VMEM((S + WP + WF, B), _f32),
        ],
        compiler_params=_CP)(graph_ins)

    final_ins = {
        'hidden': hidden,
        'smax_wt': params['smax_W'].T, 'smax_b': params['smax_b'].reshape(1, -1),
        'rec_wt': params['rec_W'].T, 'rec_b': params['rec_b'].reshape(1, -1),
    }
    logp, rec = pl.pallas_call(
        _final_body,
        out_shape=[
            jax.ShapeDtypeStruct((S, B, N_CLASSES), _f32),
            jax.ShapeDtypeStruct((S, B, IN_SIZE), _f32),
        ],
        compiler_params=_CP)(final_ins)
    return logp, rec, hidden
